# Initial kernel scaffold; baseline (speedup 1.0000x reference)
import jax
import jax.numpy as jnp
from jax.experimental import pallas as pl


def kernel(x, lengths):
    raise NotImplementedError("write your pallas kernel here")


# speedup vs baseline: 12.9135x; 12.9135x over previous
import functools

import jax
import jax.numpy as jnp
from jax import lax
from jax.experimental import pallas as pl
from jax.experimental.pallas import tpu as pltpu
from jax.experimental.pallas import tpu_sc as plsc

NC = 2
NS = 16
NW = NC * NS
LANES = 16
CH = 64
LOG2_CH = 6
INT_MAX = jnp.iinfo(jnp.int32).max


def _make_kernel(N, D, B):
  ngrp = D // LANES
  mesh = plsc.VectorSubcoreMesh(
      core_axis_name="c", subcore_axis_name="s", num_cores=NC,
      num_subcores=NS)

  @functools.partial(
      pl.kernel,
      out_type=[
          jax.ShapeDtypeStruct((B, D), jnp.float32),
          jax.ShapeDtypeStruct((B, D), jnp.int32),
      ],
      mesh=mesh,
      compiler_params=pltpu.CompilerParams(use_tc_tiling_on_sc=False),
      scratch_types=[
          pltpu.VMEM((CH, D), jnp.float32),
          pltpu.VMEM((B + 24,), jnp.int32),
          pltpu.VMEM((NW + 24,), jnp.int32),
          pltpu.VMEM((1, D), jnp.float32),
          pltpu.VMEM((1, D), jnp.int32),
      ],
  )
  def seg_pool(x_hbm, off_hbm, bnd_hbm, out_hbm, attn_hbm,
               buf, offv, bndv, ostage, istage):
    wid = lax.axis_index("s") * NC + lax.axis_index("c")
    pltpu.sync_copy(off_hbm, offv)
    pltpu.sync_copy(bnd_hbm, bndv)

    def sload(ref, i):
      return ref[pl.ds(i, LANES)][0]

    seg_lo = sload(bndv, wid)
    seg_hi = sload(bndv, wid + 1)

    def seg_body(s, _):
      pair = offv[pl.ds(s, LANES)]
      off = pair[0]
      nxt = pair[1]
      ln = nxt - off
      accs = [jnp.full((LANES,), -jnp.inf, jnp.float32) for _ in range(ngrp)]
      idxs = [jnp.full((LANES,), INT_MAX, jnp.int32) for _ in range(ngrp)]
      nwin = (ln + (CH - 1)) >> LOG2_CH

      def win_body(j, carry):
        accs, idxs = carry
        start = jnp.maximum(jnp.minimum(off + j * CH, nxt - CH), 0)
        pltpu.sync_copy(x_hbm.at[pl.ds(start, CH)], buf)
        r_lo = jnp.maximum(off - start, 0)
        r_hi = jnp.minimum(nxt - start, CH) - 1

        def row_body(r, carry):
          accs, idxs = carry
          rr = jnp.clip(r, r_lo, r_hi)
          pos = jnp.full((LANES,), start + rr - off, jnp.int32)
          naccs = []
          nidxs = []
          for k in range(ngrp):
            row = buf[rr, pl.ds(k * LANES, LANES)]
            upd = row > accs[k]
            nidxs.append(jnp.where(upd, pos, idxs[k]))
            naccs.append(jnp.where(upd, row, accs[k]))
          return naccs, nidxs

        return lax.fori_loop(0, CH, row_body, (accs, idxs))

      accs, idxs = lax.fori_loop(0, nwin, win_body, (accs, idxs))
      for k in range(ngrp):
        ostage[0, pl.ds(k * LANES, LANES)] = accs[k]
        istage[0, pl.ds(k * LANES, LANES)] = idxs[k]
      pltpu.sync_copy(ostage, out_hbm.at[pl.ds(s, 1)])
      pltpu.sync_copy(istage, attn_hbm.at[pl.ds(s, 1)])
      return 0

    lax.fori_loop(seg_lo, seg_hi, seg_body, 0)

  return seg_pool


@jax.jit
def kernel(x, lengths):
  N, D = x.shape
  B = lengths.shape[0]
  csum = jnp.cumsum(lengths, dtype=jnp.int32)
  offsets = jnp.zeros((B + 24,), jnp.int32).at[1:B + 1].set(csum)
  targets = (jnp.arange(1, NW, dtype=jnp.int32) * (N // NW)).astype(jnp.int32)
  inner = jnp.searchsorted(csum, targets, side="left").astype(jnp.int32)
  bounds = jnp.zeros((NW + 24,), jnp.int32)
  bounds = bounds.at[1:NW].set(inner).at[NW].set(B)
  out, attn = _make_kernel(N, D, B)(x, offsets, bounds)
  return (out, attn)
